# Initial kernel scaffold; baseline (speedup 1.0000x reference)
import jax
import jax.numpy as jnp
from jax.experimental import pallas as pl


def kernel(x, neigh_indices):
    raise NotImplementedError("write your pallas kernel here")


# speedup vs baseline: 10.5824x; 10.5824x over previous
import functools

import jax
import jax.numpy as jnp
from jax import lax
from jax.experimental import pallas as pl
from jax.experimental.pallas import tpu as pltpu
from jax.experimental.pallas import tpu_sc as plsc

B, H, N_HI, C = 1, 4, 163842, 32
N_LO, K = 40962, 7

NC, NS, L = 2, 16, 16
NW = NC * NS
CH = 128
CHT = 130
NIDX = CHT * K
NIDXP = 912
CPH = 320
TOTAL_CH = H * CPH
PER_W = TOTAL_CH // NW
IDX_PAD_LEN = (CPH - 1) * CH * K + NIDXP

_mesh = plsc.VectorSubcoreMesh(core_axis_name="c", subcore_axis_name="s")


@functools.partial(
    pl.kernel,
    mesh=_mesh,
    compiler_params=pltpu.CompilerParams(use_tc_tiling_on_sc=False),
    out_type=jax.ShapeDtypeStruct((H * N_LO * C,), jnp.float32),
    scratch_types=[
        pltpu.VMEM((NIDXP,), jnp.int32),
        pltpu.VMEM((NIDXP, C), jnp.float32),
        pltpu.VMEM((CHT * C,), jnp.float32),
        pltpu.SemaphoreType.DMA,
    ],
)
def _hex_pool(x_hbm, idx_hbm, out_hbm, idx_v, rows_v, out_v, sem):
    wid = lax.axis_index("s") * NC + lax.axis_index("c")

    def chunk_body(t, carry):
        c = wid * PER_W + t
        h = c // CPH
        j = c % CPH

        pltpu.sync_copy(idx_hbm.at[pl.ds(j * CH * K, NIDXP)], idx_v)

        hoff = h * N_HI

        def add_off(s, carry2):
            sl = pl.ds(s * L, L)
            idx_v[sl] = idx_v[sl] + hoff
            return carry2

        lax.fori_loop(0, NIDXP // L, add_off, 0)

        pltpu.async_copy(x_hbm.at[idx_v], rows_v, sem).wait()

        def row_body(i, carry3):
            r = i * K
            a0 = rows_v[r, pl.ds(0, L)]
            a1 = rows_v[r, pl.ds(L, L)]
            for k in range(1, K):
                a0 = jnp.maximum(a0, rows_v[r + k, pl.ds(0, L)])
                a1 = jnp.maximum(a1, rows_v[r + k, pl.ds(L, L)])
            out_v[pl.ds(i * C, L)] = a0
            out_v[pl.ds(i * C + L, L)] = a1
            return carry3

        lax.fori_loop(0, CHT, row_body, 0)

        off = (h * N_LO + j * CH) * C
        pltpu.sync_copy(out_v.at[pl.ds(0, CH * C)],
                        out_hbm.at[pl.ds(off, CH * C)])

        @pl.when(j == CPH - 1)
        def _tail():
            pltpu.sync_copy(out_v.at[pl.ds(CH * C, 2 * C)],
                            out_hbm.at[pl.ds(off + CH * C, 2 * C)])

        return carry

    lax.fori_loop(0, PER_W, chunk_body, 0)


def kernel(x, neigh_indices):
    xf = x.reshape(H * N_HI, C)
    nf = neigh_indices.astype(jnp.int32).reshape(-1)
    nf = jnp.pad(nf, (0, IDX_PAD_LEN - nf.shape[0]))
    out = _hex_pool(xf, nf)
    return out.reshape(B, H, N_LO, C)
